# baseline (speedup 1.0000x reference)
import functools

import jax
import jax.numpy as jnp
from jax import lax
from jax.experimental import pallas as pl
from jax.experimental.pallas import tpu as pltpu
from jax.experimental.pallas import tpu_sc as plsc

NUM_EMB = 1000000
D = 64
BATCH = 4096
SEQ = 200
NC = 2
NS = 16
NW = NC * NS
BBLK = BATCH // NW
NBUF = 2


def _embed_body(idxt_hbm, table_hbm, out_hbm, idx_v, rows_v, tp_v, gsems, ssems):
    wid = lax.axis_index("s") * NC + lax.axis_index("c")
    pltpu.sync_copy(idxt_hbm.at[:, pl.ds(wid * BBLK, BBLK)], idx_v)

    lane = jax.lax.iota(jnp.int32, 16)

    def fire_gather(s, b):
        pltpu.async_copy(
            table_hbm.at[idx_v.at[s, pl.ds(0, BBLK)]],
            rows_v.at[b],
            gsems[b],
        )

    def wait_gather(b):
        pltpu.make_async_copy(
            table_hbm.at[pl.ds(0, BBLK)], rows_v.at[b], gsems[b]
        ).wait()

    def transpose(b):
        def col(d, carry):
            for c in range(BBLK // 16):
                vec = plsc.load_gather(
                    rows_v.at[b],
                    [c * 16 + lane, jnp.full((16,), d, jnp.int32)],
                )
                tp_v[b, d, pl.ds(c * 16, 16)] = vec
            return carry

        lax.fori_loop(0, D, col, 0)

    def fire_store(s, b):
        for d8 in range(D // 8):
            pltpu.async_copy(
                tp_v.at[b, pl.ds(d8 * 8, 8)],
                out_hbm.at[s, d8, wid],
                ssems[b],
            )

    def wait_store_all(b):
        for d8 in range(D // 8):
            pltpu.make_async_copy(
                tp_v.at[b, pl.ds(0, 8)], out_hbm.at[0, 0, 0], ssems[b]
            ).wait()

    for b in range(NBUF):
        fire_gather(b, b)

    def pair(g, carry):
        for b in range(NBUF):
            s = g * NBUF + b
            wait_gather(b)
            transpose(b)
            fire_store(s, b)
            wait_store_all(b)
            fire_gather(s + NBUF, b)
        return carry

    lax.fori_loop(0, SEQ // NBUF - 1, pair, 0)

    for b in range(NBUF):
        s = SEQ - NBUF + b
        wait_gather(b)
        transpose(b)
        fire_store(s, b)
    for b in range(NBUF):
        wait_store_all(b)


@jax.jit
def _embed(idxt, embedding):
    mesh = plsc.VectorSubcoreMesh(
        core_axis_name="c", subcore_axis_name="s", num_cores=NC, num_subcores=NS
    )
    return pl.kernel(
        _embed_body,
        out_type=jax.ShapeDtypeStruct((SEQ, D // 8, NW, 8, BBLK), jnp.float32),
        mesh=mesh,
        scratch_types=[
            pltpu.VMEM((SEQ, BBLK), jnp.int32),
            pltpu.VMEM((NBUF, BBLK, D), jnp.float32),
            pltpu.VMEM((NBUF, D, BBLK), jnp.float32),
            [pltpu.SemaphoreType.DMA] * NBUF,
            [pltpu.SemaphoreType.DMA] * NBUF,
        ],
        compiler_params=pltpu.CompilerParams(
            use_tc_tiling_on_sc=False, needs_layout_passes=False
        ),
    )(idxt, embedding)


def kernel(inputs, embedding):
    idxt = jnp.transpose(inputs)
    out5 = _embed(idxt, embedding)
    out = out5.transpose(2, 4, 0, 1, 3).reshape(BATCH, SEQ, D)
    return out
